# baseline (speedup 1.0000x reference)
import functools

import jax
import jax.numpy as jnp
from jax import lax
from jax.experimental import pallas as pl
from jax.experimental.pallas import tpu as pltpu
from jax.experimental.pallas import tpu_sc as plsc

N_NODES = 10000
N_EDGES = 320000
D = 128
HIDDEN = 128
NUM_LAYERS = 6
ALPHA = 0.1
LN_EPS = 1e-5

NC = 2
NS = 16
NW = NC * NS
EPW = N_EDGES // NW
CH = 125
NCH = EPW // CH
CZ = 400
NZ = N_NODES // CZ
KZ = -(-NZ // NS)
ZB = 80
NZB = N_NODES // ZB
KZB = -(-NZB // NS)
DW = 128
DEG_LAG = 8

_MESH = dict(core_axis_name="c", subcore_axis_name="s", num_cores=NC,
             num_subcores=NS)


def _worker_ids():
    cid = lax.axis_index("c")
    sid = lax.axis_index("s")
    return cid, sid, cid * NS + sid


def _sc_deg_body(dst_hbm, ones_hbm, zeros_hbm, out_hbm,
                 acc, dst_v, ones_v, zero_v, sem):
    cid, sid, wid = _worker_ids()
    pltpu.sync_copy(dst_hbm.at[wid], dst_v)
    pltpu.sync_copy(ones_hbm, ones_v)
    pltpu.sync_copy(zeros_hbm, zero_v)
    for k in range(KZB):
        j = sid + NS * k

        @pl.when(j < NZB)
        def _():
            pltpu.sync_copy(zero_v, acc.at[pl.ds(j * ZB, ZB)])

    plsc.subcore_barrier()

    def step(j, carry):
        pltpu.async_copy(ones_v, acc.at[dst_v.at[j]], sem, add=True)

        @pl.when(j >= DEG_LAG)
        def _():
            pltpu.make_async_copy(ones_v, acc.at[dst_v.at[0]], sem).wait()

        return carry

    lax.fori_loop(0, NCH, step, 0)
    for _ in range(DEG_LAG):
        pltpu.make_async_copy(ones_v, acc.at[dst_v.at[0]], sem).wait()
    plsc.subcore_barrier()
    for k in range(KZ):
        j = sid + NS * k

        @pl.when(j < NZ)
        def _():
            pltpu.sync_copy(acc.at[pl.ds(j * CZ, CZ)],
                            out_hbm.at[cid, pl.ds(j * CZ, CZ)])


@functools.cache
def _sc_deg_kernel():
    return pl.kernel(
        _sc_deg_body,
        out_type=jax.ShapeDtypeStruct((NC, N_NODES, DW), jnp.float32),
        mesh=plsc.VectorSubcoreMesh(**_MESH),
        scratch_types=[
            pltpu.VMEM_SHARED((N_NODES, DW), jnp.float32),
            pltpu.VMEM((NCH, CH), jnp.int32),
            pltpu.VMEM((CH, DW), jnp.float32),
            pltpu.VMEM((ZB, DW), jnp.float32),
            pltpu.SemaphoreType.DMA,
        ],
    )


NRB = 3
NIB = 6


def _sc_scatter_body(h_hbm, ei_hbm, zeros_hbm, out_hbm, acc,
                     rowsb, idxb, gsems, ssems, isems):
    cid, sid, wid = _worker_ids()

    for c in range(NIB):
        pltpu.async_copy(ei_hbm.at[wid, c], idxb[c], isems[c])
    pltpu.sync_copy(zeros_hbm, rowsb[0].at[pl.ds(0, ZB)])
    for k in range(KZB):
        j = sid + NS * k

        @pl.when(j < NZB)
        def _():
            pltpu.sync_copy(rowsb[0].at[pl.ds(0, ZB)],
                            acc.at[pl.ds(j * ZB, ZB)])

    plsc.subcore_barrier()

    pltpu.make_async_copy(ei_hbm.at[wid, 0], idxb[0], isems[0]).wait()
    pltpu.async_copy(h_hbm.at[idxb[0].at[0]], rowsb[0], gsems[0])

    def do_chunk(j, b, b6):
        b1 = (b + 1) % NRB
        b61 = (b6 + 1) % NIB
        b64 = (b6 + 4) % NIB
        pltpu.make_async_copy(h_hbm.at[idxb[b6].at[0]], rowsb[b],
                              gsems[b]).wait()
        pltpu.async_copy(rowsb[b], acc.at[idxb[b6].at[1]], ssems[b],
                         add=True)

        @pl.when(j + 1 < NCH)
        def _():
            @pl.when(j >= 2)
            def _():
                pltpu.make_async_copy(rowsb[b1], acc.at[idxb[b61].at[1]],
                                      ssems[b1]).wait()

                @pl.when(j + 4 < NCH)
                def _():
                    pltpu.async_copy(ei_hbm.at[wid, j + 4], idxb[b64],
                                     isems[b64])

            pltpu.make_async_copy(ei_hbm.at[wid, j + 1], idxb[b61],
                                  isems[b61]).wait()
            pltpu.async_copy(h_hbm.at[idxb[b61].at[0]], rowsb[b1],
                             gsems[b1])

    UNROLL = NIB
    NT = NCH // UNROLL

    def step(t, carry):
        for u in range(UNROLL):
            do_chunk(t * UNROLL + u, u % NRB, u % NIB)
        return carry

    lax.fori_loop(0, NT, step, 0)
    for j in range(NT * UNROLL, NCH):
        do_chunk(jnp.int32(j), j % NRB, j % NIB)
    for j in range(max(0, NCH - NRB), NCH):
        b = j % NRB
        pltpu.make_async_copy(rowsb[b], acc.at[idxb[j % NIB].at[1]],
                              ssems[b]).wait()
    plsc.subcore_barrier()
    for k in range(KZ):
        j = sid + NS * k

        @pl.when(j < NZ)
        def _():
            pltpu.sync_copy(acc.at[pl.ds(j * CZ, CZ)],
                            out_hbm.at[cid, pl.ds(j * CZ, CZ)])


@functools.cache
def _sc_scatter_kernel():
    raw = pl.kernel(
        _sc_scatter_body,
        out_type=jax.ShapeDtypeStruct((NC, N_NODES, HIDDEN), jnp.float32),
        mesh=plsc.VectorSubcoreMesh(**_MESH),
        scratch_types=[
            pltpu.VMEM_SHARED((N_NODES, HIDDEN), jnp.float32),
            tuple(pltpu.VMEM((CH, HIDDEN), jnp.float32)
                  for _ in range(NRB)),
            tuple(pltpu.VMEM((2, CH), jnp.int32) for _ in range(NIB)),
            tuple(pltpu.SemaphoreType.DMA for _ in range(NRB)),
            tuple(pltpu.SemaphoreType.DMA for _ in range(NRB)),
            tuple(pltpu.SemaphoreType.DMA for _ in range(NIB)),
        ],
    )
    return raw


R = 1000
G = N_NODES // R


def _gelu(x):
    return 0.5 * x * (1.0 + lax.erf(x * (2.0 ** -0.5)))


def _ln(x, g, b):
    mu = jnp.mean(x, axis=-1, keepdims=True)
    var = jnp.mean((x - mu) ** 2, axis=-1, keepdims=True)
    return (x - mu) * lax.rsqrt(var + LN_EPS) * g + b


def _tc_in_body(x_ref, win_ref, bin_ref, gin_ref, betain_ref, degp_ref,
                w0_ref, x0_ref, hp_ref, dis_ref):
    h = jnp.dot(x_ref[...], win_ref[...], preferred_element_type=jnp.float32)
    h = _gelu(h + bin_ref[...])
    x0 = _ln(h, gin_ref[...], betain_ref[...])
    deg = degp_ref[0] + degp_ref[1]
    dis = lax.rsqrt(deg[:, 0:1] + 1.0)
    x0_ref[...] = x0
    dis_ref[...] = dis
    hp_ref[...] = jnp.dot(x0, w0_ref[...],
                          preferred_element_type=jnp.float32) * dis


def _tc_post_body(acc_ref, hp_ref, dis_ref, b_ref, g_ref, beta_ref,
                  x0_ref, cur_ref, wn_ref, cur_out_ref, hpn_ref):
    dis = dis_ref[...]
    s = acc_ref[0] + acc_ref[1] + hp_ref[...]
    out = s * dis + b_ref[...]
    out = _gelu(out)
    out = _ln(out, g_ref[...], beta_ref[...])
    out = (1.0 - ALPHA) * out + ALPHA * x0_ref[...]
    cur_new = cur_ref[...] + out
    cur_out_ref[...] = cur_new
    if hpn_ref is not None:
        hpn_ref[...] = jnp.dot(cur_new, wn_ref[...],
                               preferred_element_type=jnp.float32) * dis


def _tc_post_last_body(acc_ref, hp_ref, dis_ref, b_ref, g_ref, beta_ref,
                       x0_ref, cur_ref, cur_out_ref):
    _tc_post_body(acc_ref, hp_ref, dis_ref, b_ref, g_ref, beta_ref,
                  x0_ref, cur_ref, None, cur_out_ref, None)


_row_spec = pl.BlockSpec((R, HIDDEN), lambda i: (i, 0))
_vec_spec = pl.BlockSpec((1, HIDDEN), lambda i: (0, 0))
_w_spec = pl.BlockSpec((HIDDEN, HIDDEN), lambda i: (0, 0))
_acc_spec = pl.BlockSpec((NC, R, HIDDEN), lambda i: (0, i, 0))
_dis_spec = pl.BlockSpec((R, 1), lambda i: (i, 0))

_tc_in = pl.pallas_call(
    _tc_in_body,
    grid=(G,),
    in_specs=[_row_spec, _w_spec, _vec_spec, _vec_spec, _vec_spec,
              pl.BlockSpec((NC, R, DW), lambda i: (0, i, 0)), _w_spec],
    out_specs=[_row_spec, _row_spec, _dis_spec],
    out_shape=[
        jax.ShapeDtypeStruct((N_NODES, HIDDEN), jnp.float32),
        jax.ShapeDtypeStruct((N_NODES, HIDDEN), jnp.float32),
        jax.ShapeDtypeStruct((N_NODES, 1), jnp.float32),
    ],
)

_tc_post = pl.pallas_call(
    _tc_post_body,
    grid=(G,),
    in_specs=[_acc_spec, _row_spec, _dis_spec, _vec_spec, _vec_spec,
              _vec_spec, _row_spec, _row_spec, _w_spec],
    out_specs=[_row_spec, _row_spec],
    out_shape=[
        jax.ShapeDtypeStruct((N_NODES, HIDDEN), jnp.float32),
        jax.ShapeDtypeStruct((N_NODES, HIDDEN), jnp.float32),
    ],
)

_tc_post_last = pl.pallas_call(
    _tc_post_last_body,
    grid=(G,),
    in_specs=[_acc_spec, _row_spec, _dis_spec, _vec_spec, _vec_spec,
              _vec_spec, _row_spec, _row_spec],
    out_specs=[_row_spec],
    out_shape=[jax.ShapeDtypeStruct((N_NODES, HIDDEN), jnp.float32)],
)


def kernel(x, edge_index, W_in, b_in, g_in, beta_in, W_conv, b_conv,
           g_ln, beta_ln):
    src = edge_index[0].astype(jnp.int32).reshape(NW, NCH, CH)
    dst = edge_index[1].astype(jnp.int32).reshape(NW, NCH, CH)
    ei = jnp.stack([src, dst], axis=2)
    ones_dw = jnp.ones((CH, DW), jnp.float32)
    zeros_dw = jnp.zeros((ZB, DW), jnp.float32)
    zeros_h = jnp.zeros((ZB, HIDDEN), jnp.float32)
    row = lambda v: v.reshape(1, HIDDEN)

    degp = _sc_deg_kernel()(dst, ones_dw, zeros_dw)
    x0, hp, dis = _tc_in(x, W_in, row(b_in), row(g_in), row(beta_in),
                         degp, W_conv[0])
    cur = x0
    for i in range(NUM_LAYERS):
        acc = _sc_scatter_kernel()(hp, ei, zeros_h)
        if i + 1 < NUM_LAYERS:
            cur, hp = _tc_post(acc, hp, dis, row(b_conv[i]), row(g_ln[i]),
                               row(beta_ln[i]), x0, cur, W_conv[i + 1])
        else:
            (cur,) = _tc_post_last(acc, hp, dis, row(b_conv[i]),
                                   row(g_ln[i]), row(beta_ln[i]), x0, cur)
    return cur


# speedup vs baseline: 1.0583x; 1.0583x over previous
import functools

import jax
import jax.numpy as jnp
from jax import lax
from jax.experimental import pallas as pl
from jax.experimental.pallas import tpu as pltpu
from jax.experimental.pallas import tpu_sc as plsc

N_NODES = 10000
N_EDGES = 320000
D = 128
HIDDEN = 128
NUM_LAYERS = 6
ALPHA = 0.1
LN_EPS = 1e-5

NC = 2
NS = 16
NW = NC * NS
EPW = N_EDGES // NW
CH = 125
NCH = EPW // CH
CZ = 400
NZ = N_NODES // CZ
KZ = -(-NZ // NS)
ZB = 80
NZB = N_NODES // ZB
KZB = -(-NZB // NS)
DW = 16
DEG_LAG = 8

_MESH = dict(core_axis_name="c", subcore_axis_name="s", num_cores=NC,
             num_subcores=NS)


def _worker_ids():
    cid = lax.axis_index("c")
    sid = lax.axis_index("s")
    return cid, sid, cid * NS + sid


def _sc_deg_body(dst_hbm, ones_hbm, zeros_hbm, out_hbm,
                 acc, dst_v, ones_v, zero_v, sem):
    cid, sid, wid = _worker_ids()
    pltpu.sync_copy(dst_hbm.at[wid], dst_v)
    pltpu.sync_copy(ones_hbm, ones_v)
    pltpu.sync_copy(zeros_hbm, zero_v)
    for k in range(KZB):
        j = sid + NS * k

        @pl.when(j < NZB)
        def _():
            pltpu.sync_copy(zero_v, acc.at[pl.ds(j * ZB, ZB)])

    plsc.subcore_barrier()

    def step(j, carry):
        pltpu.async_copy(ones_v, acc.at[dst_v.at[j]], sem, add=True)

        @pl.when(j >= DEG_LAG)
        def _():
            pltpu.make_async_copy(ones_v, acc.at[dst_v.at[0]], sem).wait()

        return carry

    lax.fori_loop(0, NCH, step, 0)
    for _ in range(DEG_LAG):
        pltpu.make_async_copy(ones_v, acc.at[dst_v.at[0]], sem).wait()
    plsc.subcore_barrier()
    for k in range(KZ):
        j = sid + NS * k

        @pl.when(j < NZ)
        def _():
            pltpu.sync_copy(acc.at[pl.ds(j * CZ, CZ)],
                            out_hbm.at[cid, pl.ds(j * CZ, CZ)])


@functools.cache
def _sc_deg_kernel():
    return pl.kernel(
        _sc_deg_body,
        out_type=jax.ShapeDtypeStruct((NC, N_NODES, DW), jnp.float32),
        mesh=plsc.VectorSubcoreMesh(**_MESH),
        compiler_params=pltpu.CompilerParams(use_tc_tiling_on_sc=False),
        scratch_types=[
            pltpu.VMEM_SHARED((N_NODES, DW), jnp.float32),
            pltpu.VMEM((NCH, CH), jnp.int32),
            pltpu.VMEM((CH, DW), jnp.float32),
            pltpu.VMEM((ZB, DW), jnp.float32),
            pltpu.SemaphoreType.DMA,
        ],
    )


NRB = 3
NIB = 6


def _sc_scatter_body(h_hbm, ei_hbm, zeros_hbm, out_hbm, acc,
                     rowsb, idxb, gsems, ssems, isems):
    cid, sid, wid = _worker_ids()

    for c in range(NIB):
        pltpu.async_copy(ei_hbm.at[wid, c], idxb[c], isems[c])
    pltpu.sync_copy(zeros_hbm, rowsb[0].at[pl.ds(0, ZB)])
    for k in range(KZB):
        j = sid + NS * k

        @pl.when(j < NZB)
        def _():
            pltpu.sync_copy(rowsb[0].at[pl.ds(0, ZB)],
                            acc.at[pl.ds(j * ZB, ZB)])

    plsc.subcore_barrier()

    pltpu.make_async_copy(ei_hbm.at[wid, 0], idxb[0], isems[0]).wait()
    pltpu.async_copy(h_hbm.at[idxb[0].at[0]], rowsb[0], gsems[0])

    def do_chunk(j, b, b6):
        b1 = (b + 1) % NRB
        b61 = (b6 + 1) % NIB
        b64 = (b6 + 4) % NIB
        pltpu.make_async_copy(h_hbm.at[idxb[b6].at[0]], rowsb[b],
                              gsems[b]).wait()
        pltpu.async_copy(rowsb[b], acc.at[idxb[b6].at[1]], ssems[b],
                         add=True)

        @pl.when(j + 1 < NCH)
        def _():
            @pl.when(j >= 2)
            def _():
                pltpu.make_async_copy(rowsb[b1], acc.at[idxb[b61].at[1]],
                                      ssems[b1]).wait()

                @pl.when(j + 4 < NCH)
                def _():
                    pltpu.async_copy(ei_hbm.at[wid, j + 4], idxb[b64],
                                     isems[b64])

            pltpu.make_async_copy(ei_hbm.at[wid, j + 1], idxb[b61],
                                  isems[b61]).wait()
            pltpu.async_copy(h_hbm.at[idxb[b61].at[0]], rowsb[b1],
                             gsems[b1])

    UNROLL = NIB
    NT = NCH // UNROLL

    def step(t, carry):
        for u in range(UNROLL):
            do_chunk(t * UNROLL + u, u % NRB, u % NIB)
        return carry

    lax.fori_loop(0, NT, step, 0)
    for j in range(NT * UNROLL, NCH):
        do_chunk(jnp.int32(j), j % NRB, j % NIB)
    for j in range(max(0, NCH - NRB), NCH):
        b = j % NRB
        pltpu.make_async_copy(rowsb[b], acc.at[idxb[j % NIB].at[1]],
                              ssems[b]).wait()
    plsc.subcore_barrier()
    for k in range(KZ):
        j = sid + NS * k

        @pl.when(j < NZ)
        def _():
            pltpu.sync_copy(acc.at[pl.ds(j * CZ, CZ)],
                            out_hbm.at[cid, pl.ds(j * CZ, CZ)])


@functools.cache
def _sc_scatter_kernel():
    raw = pl.kernel(
        _sc_scatter_body,
        out_type=jax.ShapeDtypeStruct((NC, N_NODES, HIDDEN), jnp.float32),
        mesh=plsc.VectorSubcoreMesh(**_MESH),
        scratch_types=[
            pltpu.VMEM_SHARED((N_NODES, HIDDEN), jnp.float32),
            tuple(pltpu.VMEM((CH, HIDDEN), jnp.float32)
                  for _ in range(NRB)),
            tuple(pltpu.VMEM((2, CH), jnp.int32) for _ in range(NIB)),
            tuple(pltpu.SemaphoreType.DMA for _ in range(NRB)),
            tuple(pltpu.SemaphoreType.DMA for _ in range(NRB)),
            tuple(pltpu.SemaphoreType.DMA for _ in range(NIB)),
        ],
    )
    return raw


R = 1000
G = N_NODES // R


def _gelu(x):
    return 0.5 * x * (1.0 + lax.erf(x * (2.0 ** -0.5)))


def _ln(x, g, b):
    mu = jnp.mean(x, axis=-1, keepdims=True)
    var = jnp.mean((x - mu) ** 2, axis=-1, keepdims=True)
    return (x - mu) * lax.rsqrt(var + LN_EPS) * g + b


def _tc_in_body(x_ref, win_ref, bin_ref, gin_ref, betain_ref, degp_ref,
                w0_ref, x0_ref, hp_ref, dis_ref):
    h = jnp.dot(x_ref[...], win_ref[...], preferred_element_type=jnp.float32)
    h = _gelu(h + bin_ref[...])
    x0 = _ln(h, gin_ref[...], betain_ref[...])
    deg = degp_ref[0] + degp_ref[1]
    dis = lax.rsqrt(deg[:, 0:1] + 1.0)
    x0_ref[...] = x0
    dis_ref[...] = dis
    hp_ref[...] = jnp.dot(x0, w0_ref[...],
                          preferred_element_type=jnp.float32) * dis


def _tc_post_body(acc_ref, hp_ref, dis_ref, b_ref, g_ref, beta_ref,
                  x0_ref, cur_ref, wn_ref, cur_out_ref, hpn_ref):
    dis = dis_ref[...]
    s = acc_ref[0] + acc_ref[1] + hp_ref[...]
    out = s * dis + b_ref[...]
    out = _gelu(out)
    out = _ln(out, g_ref[...], beta_ref[...])
    out = (1.0 - ALPHA) * out + ALPHA * x0_ref[...]
    cur_new = cur_ref[...] + out
    cur_out_ref[...] = cur_new
    if hpn_ref is not None:
        hpn_ref[...] = jnp.dot(cur_new, wn_ref[...],
                               preferred_element_type=jnp.float32) * dis


def _tc_post_last_body(acc_ref, hp_ref, dis_ref, b_ref, g_ref, beta_ref,
                       x0_ref, cur_ref, cur_out_ref):
    _tc_post_body(acc_ref, hp_ref, dis_ref, b_ref, g_ref, beta_ref,
                  x0_ref, cur_ref, None, cur_out_ref, None)


_row_spec = pl.BlockSpec((R, HIDDEN), lambda i: (i, 0))
_vec_spec = pl.BlockSpec((1, HIDDEN), lambda i: (0, 0))
_w_spec = pl.BlockSpec((HIDDEN, HIDDEN), lambda i: (0, 0))
_acc_spec = pl.BlockSpec((NC, R, HIDDEN), lambda i: (0, i, 0))
_dis_spec = pl.BlockSpec((R, 1), lambda i: (i, 0))

_tc_in = pl.pallas_call(
    _tc_in_body,
    grid=(G,),
    in_specs=[_row_spec, _w_spec, _vec_spec, _vec_spec, _vec_spec,
              pl.BlockSpec((NC, R, DW), lambda i: (0, i, 0)), _w_spec],
    out_specs=[_row_spec, _row_spec, _dis_spec],
    out_shape=[
        jax.ShapeDtypeStruct((N_NODES, HIDDEN), jnp.float32),
        jax.ShapeDtypeStruct((N_NODES, HIDDEN), jnp.float32),
        jax.ShapeDtypeStruct((N_NODES, 1), jnp.float32),
    ],
)

_tc_post = pl.pallas_call(
    _tc_post_body,
    grid=(G,),
    in_specs=[_acc_spec, _row_spec, _dis_spec, _vec_spec, _vec_spec,
              _vec_spec, _row_spec, _row_spec, _w_spec],
    out_specs=[_row_spec, _row_spec],
    out_shape=[
        jax.ShapeDtypeStruct((N_NODES, HIDDEN), jnp.float32),
        jax.ShapeDtypeStruct((N_NODES, HIDDEN), jnp.float32),
    ],
)

_tc_post_last = pl.pallas_call(
    _tc_post_last_body,
    grid=(G,),
    in_specs=[_acc_spec, _row_spec, _dis_spec, _vec_spec, _vec_spec,
              _vec_spec, _row_spec, _row_spec],
    out_specs=[_row_spec],
    out_shape=[jax.ShapeDtypeStruct((N_NODES, HIDDEN), jnp.float32)],
)


def kernel(x, edge_index, W_in, b_in, g_in, beta_in, W_conv, b_conv,
           g_ln, beta_ln):
    src = edge_index[0].astype(jnp.int32).reshape(NW, NCH, CH)
    dst = edge_index[1].astype(jnp.int32).reshape(NW, NCH, CH)
    ei = jnp.stack([src, dst], axis=2)
    ones_dw = jnp.ones((CH, DW), jnp.float32)
    zeros_dw = jnp.zeros((ZB, DW), jnp.float32)
    zeros_h = jnp.zeros((ZB, HIDDEN), jnp.float32)
    row = lambda v: v.reshape(1, HIDDEN)

    degp = _sc_deg_kernel()(dst, ones_dw, zeros_dw)
    x0, hp, dis = _tc_in(x, W_in, row(b_in), row(g_in), row(beta_in),
                         degp, W_conv[0])
    cur = x0
    for i in range(NUM_LAYERS):
        acc = _sc_scatter_kernel()(hp, ei, zeros_h)
        if i + 1 < NUM_LAYERS:
            cur, hp = _tc_post(acc, hp, dis, row(b_conv[i]), row(g_ln[i]),
                               row(beta_ln[i]), x0, cur, W_conv[i + 1])
        else:
            (cur,) = _tc_post_last(acc, hp, dis, row(b_conv[i]),
                                   row(g_ln[i]), row(beta_ln[i]), x0, cur)
    return cur


# speedup vs baseline: 1.0732x; 1.0141x over previous
import functools

import jax
import jax.numpy as jnp
from jax import lax
from jax.experimental import pallas as pl
from jax.experimental.pallas import tpu as pltpu
from jax.experimental.pallas import tpu_sc as plsc

N_NODES = 10000
N_EDGES = 320000
D = 128
HIDDEN = 128
NUM_LAYERS = 6
ALPHA = 0.1
LN_EPS = 1e-5

NC = 2
NS = 16
NW = NC * NS
EPW = N_EDGES // NW
CH = 125
NCH = EPW // CH
CZ = 400
NZ = N_NODES // CZ
KZ = -(-NZ // NS)
ZB = 80
NZB = N_NODES // ZB
KZB = -(-NZB // NS)
DW = 16
DEG_LAG = 8

_MESH = dict(core_axis_name="c", subcore_axis_name="s", num_cores=NC,
             num_subcores=NS)


def _worker_ids():
    cid = lax.axis_index("c")
    sid = lax.axis_index("s")
    return cid, sid, cid * NS + sid


def _sc_deg_body(dst_hbm, ones_hbm, zeros_hbm, out_hbm,
                 acc, dst_v, ones_v, zero_v, sem):
    cid, sid, wid = _worker_ids()
    pltpu.sync_copy(dst_hbm.at[wid], dst_v)
    pltpu.sync_copy(ones_hbm, ones_v)
    pltpu.sync_copy(zeros_hbm, zero_v)
    for k in range(KZB):
        j = sid + NS * k

        @pl.when(j < NZB)
        def _():
            pltpu.sync_copy(zero_v, acc.at[pl.ds(j * ZB, ZB)])

    plsc.subcore_barrier()

    def step(j, carry):
        pltpu.async_copy(ones_v, acc.at[dst_v.at[j]], sem, add=True)

        @pl.when(j >= DEG_LAG)
        def _():
            pltpu.make_async_copy(ones_v, acc.at[dst_v.at[0]], sem).wait()

        return carry

    lax.fori_loop(0, NCH, step, 0)
    for _ in range(DEG_LAG):
        pltpu.make_async_copy(ones_v, acc.at[dst_v.at[0]], sem).wait()
    plsc.subcore_barrier()
    for k in range(KZ):
        j = sid + NS * k

        @pl.when(j < NZ)
        def _():
            pltpu.sync_copy(acc.at[pl.ds(j * CZ, CZ)],
                            out_hbm.at[cid, pl.ds(j * CZ, CZ)])


@functools.cache
def _sc_deg_kernel():
    return pl.kernel(
        _sc_deg_body,
        out_type=jax.ShapeDtypeStruct((NC, N_NODES, DW), jnp.float32),
        mesh=plsc.VectorSubcoreMesh(**_MESH),
        compiler_params=pltpu.CompilerParams(use_tc_tiling_on_sc=False),
        scratch_types=[
            pltpu.VMEM_SHARED((N_NODES, DW), jnp.float32),
            pltpu.VMEM((NCH, CH), jnp.int32),
            pltpu.VMEM((CH, DW), jnp.float32),
            pltpu.VMEM((ZB, DW), jnp.float32),
            pltpu.SemaphoreType.DMA,
        ],
    )


NRB = 3
NIB = 6


def _sc_scatter_body(h_hbm, ei_hbm, zeros_hbm, out_hbm, acc,
                     rowsb, idxb, gsems, ssems, isems):
    cid, sid, wid = _worker_ids()

    for c in range(NIB):
        pltpu.async_copy(ei_hbm.at[wid, c], idxb[c], isems[c])
    pltpu.make_async_copy(ei_hbm.at[wid, 1], idxb[1], isems[1]).wait()
    pltpu.async_copy(h_hbm.at[idxb[1].at[0]], rowsb[1], gsems[1])
    pltpu.make_async_copy(ei_hbm.at[wid, 2], idxb[2], isems[2]).wait()
    pltpu.async_copy(h_hbm.at[idxb[2].at[0]], rowsb[2], gsems[2])
    pltpu.sync_copy(zeros_hbm, rowsb[0].at[pl.ds(0, ZB)])
    for k in range(KZB):
        j = sid + NS * k

        @pl.when(j < NZB)
        def _():
            pltpu.sync_copy(rowsb[0].at[pl.ds(0, ZB)],
                            acc.at[pl.ds(j * ZB, ZB)])

    pltpu.make_async_copy(ei_hbm.at[wid, 0], idxb[0], isems[0]).wait()
    pltpu.async_copy(h_hbm.at[idxb[0].at[0]], rowsb[0], gsems[0])
    plsc.subcore_barrier()

    def do_chunk(j, b, b6):
        b1 = (b + 1) % NRB
        b61 = (b6 + 1) % NIB
        b64 = (b6 + 4) % NIB
        pltpu.make_async_copy(h_hbm.at[idxb[b6].at[0]], rowsb[b],
                              gsems[b]).wait()
        pltpu.async_copy(rowsb[b], acc.at[idxb[b6].at[1]], ssems[b],
                         add=True)

        @pl.when(jnp.logical_and(j + 1 < NCH, j >= 2))
        def _():
            pltpu.make_async_copy(rowsb[b1], acc.at[idxb[b61].at[1]],
                                  ssems[b1]).wait()

            @pl.when(j + 4 < NCH)
            def _():
                pltpu.async_copy(ei_hbm.at[wid, j + 4], idxb[b64],
                                 isems[b64])

            pltpu.make_async_copy(ei_hbm.at[wid, j + 1], idxb[b61],
                                  isems[b61]).wait()
            pltpu.async_copy(h_hbm.at[idxb[b61].at[0]], rowsb[b1],
                             gsems[b1])

    UNROLL = NIB
    NT = NCH // UNROLL

    def step(t, carry):
        for u in range(UNROLL):
            do_chunk(t * UNROLL + u, u % NRB, u % NIB)
        return carry

    lax.fori_loop(0, NT, step, 0)
    for j in range(NT * UNROLL, NCH):
        do_chunk(jnp.int32(j), j % NRB, j % NIB)
    for j in range(max(0, NCH - NRB), NCH):
        b = j % NRB
        pltpu.make_async_copy(rowsb[b], acc.at[idxb[j % NIB].at[1]],
                              ssems[b]).wait()
    plsc.subcore_barrier()
    for k in range(KZ):
        j = sid + NS * k

        @pl.when(j < NZ)
        def _():
            pltpu.sync_copy(acc.at[pl.ds(j * CZ, CZ)],
                            out_hbm.at[cid, pl.ds(j * CZ, CZ)])


@functools.cache
def _sc_scatter_kernel():
    raw = pl.kernel(
        _sc_scatter_body,
        out_type=jax.ShapeDtypeStruct((NC, N_NODES, HIDDEN), jnp.float32),
        mesh=plsc.VectorSubcoreMesh(**_MESH),
        scratch_types=[
            pltpu.VMEM_SHARED((N_NODES, HIDDEN), jnp.float32),
            tuple(pltpu.VMEM((CH, HIDDEN), jnp.float32)
                  for _ in range(NRB)),
            tuple(pltpu.VMEM((2, CH), jnp.int32) for _ in range(NIB)),
            tuple(pltpu.SemaphoreType.DMA for _ in range(NRB)),
            tuple(pltpu.SemaphoreType.DMA for _ in range(NRB)),
            tuple(pltpu.SemaphoreType.DMA for _ in range(NIB)),
        ],
    )
    return raw


R = 2000
G = N_NODES // R


def _gelu(x):
    return 0.5 * x * (1.0 + lax.erf(x * (2.0 ** -0.5)))


def _ln(x, g, b):
    mu = jnp.mean(x, axis=-1, keepdims=True)
    var = jnp.mean((x - mu) ** 2, axis=-1, keepdims=True)
    return (x - mu) * lax.rsqrt(var + LN_EPS) * g + b


def _tc_in_body(x_ref, win_ref, bin_ref, gin_ref, betain_ref, degp_ref,
                w0_ref, x0_ref, hp_ref, dis_ref):
    h = jnp.dot(x_ref[...], win_ref[...], preferred_element_type=jnp.float32)
    h = _gelu(h + bin_ref[...])
    x0 = _ln(h, gin_ref[...], betain_ref[...])
    deg = degp_ref[0] + degp_ref[1]
    dis = lax.rsqrt(deg[:, 0:1] + 1.0)
    x0_ref[...] = x0
    dis_ref[...] = dis
    hp_ref[...] = jnp.dot(x0, w0_ref[...],
                          preferred_element_type=jnp.float32) * dis


def _tc_post_body(acc_ref, hp_ref, dis_ref, b_ref, g_ref, beta_ref,
                  x0_ref, cur_ref, wn_ref, cur_out_ref, hpn_ref):
    dis = dis_ref[...]
    s = acc_ref[0] + acc_ref[1] + hp_ref[...]
    out = s * dis + b_ref[...]
    out = _gelu(out)
    out = _ln(out, g_ref[...], beta_ref[...])
    out = (1.0 - ALPHA) * out + ALPHA * x0_ref[...]
    cur_new = cur_ref[...] + out
    cur_out_ref[...] = cur_new
    if hpn_ref is not None:
        hpn_ref[...] = jnp.dot(cur_new, wn_ref[...],
                               preferred_element_type=jnp.float32) * dis


def _tc_post_last_body(acc_ref, hp_ref, dis_ref, b_ref, g_ref, beta_ref,
                       x0_ref, cur_ref, cur_out_ref):
    _tc_post_body(acc_ref, hp_ref, dis_ref, b_ref, g_ref, beta_ref,
                  x0_ref, cur_ref, None, cur_out_ref, None)


_row_spec = pl.BlockSpec((R, HIDDEN), lambda i: (i, 0))
_vec_spec = pl.BlockSpec((1, HIDDEN), lambda i: (0, 0))
_w_spec = pl.BlockSpec((HIDDEN, HIDDEN), lambda i: (0, 0))
_acc_spec = pl.BlockSpec((NC, R, HIDDEN), lambda i: (0, i, 0))
_dis_spec = pl.BlockSpec((R, 1), lambda i: (i, 0))

_tc_in = pl.pallas_call(
    _tc_in_body,
    grid=(G,),
    in_specs=[_row_spec, _w_spec, _vec_spec, _vec_spec, _vec_spec,
              pl.BlockSpec((NC, R, DW), lambda i: (0, i, 0)), _w_spec],
    out_specs=[_row_spec, _row_spec, _dis_spec],
    out_shape=[
        jax.ShapeDtypeStruct((N_NODES, HIDDEN), jnp.float32),
        jax.ShapeDtypeStruct((N_NODES, HIDDEN), jnp.float32),
        jax.ShapeDtypeStruct((N_NODES, 1), jnp.float32),
    ],
)

_tc_post = pl.pallas_call(
    _tc_post_body,
    grid=(G,),
    in_specs=[_acc_spec, _row_spec, _dis_spec, _vec_spec, _vec_spec,
              _vec_spec, _row_spec, _row_spec, _w_spec],
    out_specs=[_row_spec, _row_spec],
    out_shape=[
        jax.ShapeDtypeStruct((N_NODES, HIDDEN), jnp.float32),
        jax.ShapeDtypeStruct((N_NODES, HIDDEN), jnp.float32),
    ],
)

_tc_post_last = pl.pallas_call(
    _tc_post_last_body,
    grid=(G,),
    in_specs=[_acc_spec, _row_spec, _dis_spec, _vec_spec, _vec_spec,
              _vec_spec, _row_spec, _row_spec],
    out_specs=[_row_spec],
    out_shape=[jax.ShapeDtypeStruct((N_NODES, HIDDEN), jnp.float32)],
)


def kernel(x, edge_index, W_in, b_in, g_in, beta_in, W_conv, b_conv,
           g_ln, beta_ln):
    src = edge_index[0].astype(jnp.int32).reshape(NW, NCH, CH)
    dst = edge_index[1].astype(jnp.int32).reshape(NW, NCH, CH)
    ei = jnp.stack([src, dst], axis=2)
    ones_dw = jnp.ones((CH, DW), jnp.float32)
    zeros_dw = jnp.zeros((ZB, DW), jnp.float32)
    zeros_h = jnp.zeros((ZB, HIDDEN), jnp.float32)
    row = lambda v: v.reshape(1, HIDDEN)

    degp = _sc_deg_kernel()(dst, ones_dw, zeros_dw)
    x0, hp, dis = _tc_in(x, W_in, row(b_in), row(g_in), row(beta_in),
                         degp, W_conv[0])
    cur = x0
    for i in range(NUM_LAYERS):
        acc = _sc_scatter_kernel()(hp, ei, zeros_h)
        if i + 1 < NUM_LAYERS:
            cur, hp = _tc_post(acc, hp, dis, row(b_conv[i]), row(g_ln[i]),
                               row(beta_ln[i]), x0, cur, W_conv[i + 1])
        else:
            (cur,) = _tc_post_last(acc, hp, dis, row(b_conv[i]),
                                   row(g_ln[i]), row(beta_ln[i]), x0, cur)
    return cur
